# baseline (speedup 1.0000x reference)
import functools

import jax
import jax.numpy as jnp
from jax import lax
from jax.experimental import pallas as pl
from jax.experimental.pallas import tpu as pltpu
from jax.experimental.pallas import tpu_sc as plsc

N = 10000
E = 320000
D = 128
NC, NS, L = 2, 16, 16
NW = NC * NS
NP = 10240
CH = 128
EPW = 10240
NCH = EPW // CH
EPAD = NW * EPW
TCH = EPAD // CH
NBUF = 2
NSS = NCH // NBUF
RPT = NP // NS
K0 = 156
K1 = (TCH - NS * K0) // NS

_mesh = plsc.VectorSubcoreMesh(core_axis_name="c", subcore_axis_name="s")


@functools.partial(
    pl.kernel,
    out_type=jax.ShapeDtypeStruct((NW, NP), jnp.float32),
    mesh=_mesh,
    compiler_params=pltpu.CompilerParams(needs_layout_passes=False),
    scratch_types=[
        pltpu.VMEM((NCH, CH), jnp.int32),
        pltpu.VMEM((NP,), jnp.float32),
    ],
)
def _sc_degrees(src_hbm, out_hbm, idx_v, deg_v):
    w = lax.axis_index("s") * NC + lax.axis_index("c")
    pltpu.sync_copy(src_hbm.at[pl.ds(w * NCH, NCH)], idx_v)
    zeros = jnp.zeros((L,), jnp.float32)
    ones = jnp.ones((L,), jnp.float32)

    def zbody(i, carry):
        deg_v[pl.ds(i * L, L)] = zeros
        return carry

    lax.fori_loop(0, NP // L, zbody, 0)

    def body(r, carry):
        for c in range(CH // L):
            idx = idx_v[r, pl.ds(c * L, L)]
            plsc.addupdate_scatter(deg_v, [idx], ones)
        return carry

    lax.fori_loop(0, NCH, body, 0)
    pltpu.sync_copy(deg_v, out_hbm.at[w])


def _tc_project(featp, degs_part, W):
    def body(f_ref, d_ref, w_ref, o_ref):
        deg = jnp.sum(d_ref[...], axis=0)
        nrm = lax.rsqrt(jnp.maximum(deg, 1.0))
        o_ref[...] = lax.dot_general(
            f_ref[...] * nrm[:, None], w_ref[...],
            (((1,), (1,)), ((), ())), preferred_element_type=jnp.float32)

    RB = 1024
    return pl.pallas_call(
        body,
        grid=(NP // RB,),
        in_specs=[
            pl.BlockSpec((RB, D), lambda i: (i, 0)),
            pl.BlockSpec((NW, RB), lambda i: (0, i)),
            pl.BlockSpec((D, D), lambda i: (0, 0)),
        ],
        out_specs=pl.BlockSpec((RB, D), lambda i: (i, 0)),
        out_shape=jax.ShapeDtypeStruct((NP, D), jnp.float32),
    )(featp, degs_part, W)


@functools.partial(
    pl.kernel,
    out_type=jax.ShapeDtypeStruct((NC, NP, D), jnp.float32),
    mesh=_mesh,
    compiler_params=pltpu.CompilerParams(needs_layout_passes=False),
    scratch_types=[
        pltpu.VMEM((2, NBUF, CH), jnp.int32),
        pltpu.VMEM((2, NBUF, CH), jnp.int32),
        pltpu.VMEM((NBUF, CH, D), jnp.float32),
        pltpu.VMEM_SHARED((NP, D), jnp.float32),
        pltpu.SemaphoreType.DMA((2,)),
        pltpu.SemaphoreType.DMA((NBUF,)),
        pltpu.SemaphoreType.DMA((NBUF,)),
    ],
)
def _sc_aggregate(g_hbm, src_hbm, dst_hbm, out_hbm, si_v, di_v, rows_v, h_sh,
                  isem, gsem, ssem):
    c = lax.axis_index("c")
    s = lax.axis_index("s")
    base = lax.select(c == 0, s * K0,
                      jnp.minimum(NS * K0 + s * K1, TCH - NBUF))
    nss2 = lax.select(c == 0, K0 // (2 * NBUF), K1 // (2 * NBUF))

    def fetch_idx(t, slot):
        a = pltpu.async_copy(
            src_hbm.at[pl.ds(base + t * NBUF, NBUF)], si_v.at[slot],
            isem.at[slot])
        b = pltpu.async_copy(
            dst_hbm.at[pl.ds(base + t * NBUF, NBUF)], di_v.at[slot],
            isem.at[slot])
        return a, b

    i0 = fetch_idx(0, 0)

    with jax.named_scope("agg_zero"):
        zeros = jnp.zeros((L,), jnp.float32)

        def zbody(i, carry):
            for cc in range(D // L):
                rows_v[0, i, pl.ds(cc * L, L)] = zeros
            return carry

        lax.fori_loop(0, CH, zbody, 0)
        for k in range(RPT // CH):
            pltpu.sync_copy(rows_v.at[0], h_sh.at[pl.ds(s * RPT + k * CH, CH)])
        plsc.subcore_barrier()
        for d in i0:
            d.wait()

    def step(t2, carry):
        for slot in range(2):
            t = t2 * 2 + slot
            nxt = 1 - slot

            @pl.when(t + 1 < 2 * nss2)
            def _():
                fetch_idx(t + 1, nxt)

            gds = []
            for b in range(NBUF):
                gds.append(pltpu.async_copy(
                    g_hbm.at[si_v.at[slot, b]], rows_v.at[b], gsem.at[b]))
            sds = []
            for b in range(NBUF):
                gds[b].wait()
                sds.append(pltpu.async_copy(
                    rows_v.at[b], h_sh.at[di_v.at[slot, b]], ssem.at[b],
                    add=True))
            for d in sds:
                d.wait()

            @pl.when(t + 1 < 2 * nss2)
            def _():
                pltpu.make_async_copy(
                    src_hbm.at[pl.ds(0, NBUF)], si_v.at[nxt],
                    isem.at[nxt]).wait()
                pltpu.make_async_copy(
                    dst_hbm.at[pl.ds(0, NBUF)], di_v.at[nxt],
                    isem.at[nxt]).wait()
        return carry

    with jax.named_scope("agg_main"):
        lax.fori_loop(0, nss2, step, 0)
        plsc.subcore_barrier()

    with jax.named_scope("agg_wb"):
        pltpu.sync_copy(h_sh.at[pl.ds(s * RPT, RPT)],
                        out_hbm.at[c, pl.ds(s * RPT, RPT)])


def _tc_finish(hpart, W, b2):
    def body(p_ref, w_ref, b_ref, o_ref):
        bias2 = b_ref[...] + jnp.sum(w_ref[...], axis=1)[None, :]
        o_ref[...] = p_ref[0] + p_ref[1] + bias2

    RB = 1000
    return pl.pallas_call(
        body,
        grid=(N // RB,),
        in_specs=[
            pl.BlockSpec((NC, RB, D), lambda i: (0, i, 0)),
            pl.BlockSpec((D, D), lambda i: (0, 0)),
            pl.BlockSpec((1, D), lambda i: (0, 0)),
        ],
        out_specs=pl.BlockSpec((RB, D), lambda i: (i, 0)),
        out_shape=jax.ShapeDtypeStruct((N, D), jnp.float32),
    )(hpart, W, b2)


def kernel(feature, edge_index, W_lin, b_lin):
    src = edge_index[0].astype(jnp.int32)
    dst = edge_index[1].astype(jnp.int32)
    pad = jnp.full((EPAD - E,), N, dtype=jnp.int32)
    srcc = jnp.concatenate([src, pad]).reshape(TCH, CH)
    dstc = jnp.concatenate([dst, pad]).reshape(TCH, CH)
    featp = jnp.pad(feature, ((0, NP - N), (0, 0)))

    degs_part = _sc_degrees(srcc)
    g = _tc_project(featp, degs_part, W_lin)
    hpart = _sc_aggregate(g, srcc, dstc)
    return _tc_finish(hpart, W_lin, jnp.reshape(b_lin, (1, D)))


# speedup vs baseline: 1.0745x; 1.0745x over previous
import functools

import jax
import jax.numpy as jnp
from jax import lax
from jax.experimental import pallas as pl
from jax.experimental.pallas import tpu as pltpu
from jax.experimental.pallas import tpu_sc as plsc

N = 10000
E = 320000
D = 128
NC, NS, L = 2, 16, 16
NW = NC * NS
NP = 10240
CH = 128
EPW = 10240
NCH = EPW // CH
EPAD = NW * EPW
TCH = EPAD // CH
NBUF = 2
NSS = NCH // NBUF
RPT = NP // NS
K0 = 148
K1 = (TCH - NS * K0) // NS

_mesh = plsc.VectorSubcoreMesh(core_axis_name="c", subcore_axis_name="s")


@functools.partial(
    pl.kernel,
    out_type=jax.ShapeDtypeStruct((NW, NP), jnp.float32),
    mesh=_mesh,
    compiler_params=pltpu.CompilerParams(needs_layout_passes=False),
    scratch_types=[
        pltpu.VMEM((NCH, CH), jnp.int32),
        pltpu.VMEM((NP,), jnp.float32),
    ],
)
def _sc_degrees(src_hbm, out_hbm, idx_v, deg_v):
    w = lax.axis_index("s") * NC + lax.axis_index("c")
    pltpu.sync_copy(src_hbm.at[pl.ds(w * NCH, NCH)], idx_v)
    zeros = jnp.zeros((L,), jnp.float32)
    ones = jnp.ones((L,), jnp.float32)

    def zbody(i, carry):
        deg_v[pl.ds(i * L, L)] = zeros
        return carry

    lax.fori_loop(0, NP // L, zbody, 0)

    def body(r, carry):
        for c in range(CH // L):
            idx = idx_v[r, pl.ds(c * L, L)]
            plsc.addupdate_scatter(deg_v, [idx], ones)
        return carry

    lax.fori_loop(0, NCH, body, 0)
    pltpu.sync_copy(deg_v, out_hbm.at[w])


def _tc_project(featp, degs_part, W):
    def body(f_ref, d_ref, w_ref, o_ref):
        deg = jnp.sum(d_ref[...], axis=0)
        nrm = lax.rsqrt(jnp.maximum(deg, 1.0))
        o_ref[...] = lax.dot_general(
            f_ref[...] * nrm[:, None], w_ref[...],
            (((1,), (1,)), ((), ())), preferred_element_type=jnp.float32)

    RB = 1024
    return pl.pallas_call(
        body,
        grid=(NP // RB,),
        in_specs=[
            pl.BlockSpec((RB, D), lambda i: (i, 0)),
            pl.BlockSpec((NW, RB), lambda i: (0, i)),
            pl.BlockSpec((D, D), lambda i: (0, 0)),
        ],
        out_specs=pl.BlockSpec((RB, D), lambda i: (i, 0)),
        out_shape=jax.ShapeDtypeStruct((NP, D), jnp.float32),
    )(featp, degs_part, W)


@functools.partial(
    pl.kernel,
    out_type=jax.ShapeDtypeStruct((NC, NP, D), jnp.float32),
    mesh=_mesh,
    compiler_params=pltpu.CompilerParams(needs_layout_passes=False),
    scratch_types=[
        pltpu.VMEM((2, NBUF, CH), jnp.int32),
        pltpu.VMEM((2, NBUF, CH), jnp.int32),
        pltpu.VMEM((NBUF, CH, D), jnp.float32),
        pltpu.VMEM_SHARED((NP, D), jnp.float32),
        pltpu.SemaphoreType.DMA((2,)),
        pltpu.SemaphoreType.DMA((NBUF,)),
        pltpu.SemaphoreType.DMA((NBUF,)),
    ],
)
def _sc_aggregate(g_hbm, src_hbm, dst_hbm, out_hbm, si_v, di_v, rows_v, h_sh,
                  isem, gsem, ssem):
    c = lax.axis_index("c")
    s = lax.axis_index("s")
    base = lax.select(c == 0, s * K0,
                      jnp.minimum(NS * K0 + s * K1, TCH - NBUF))
    nss2 = lax.select(c == 0, K0 // (2 * NBUF), K1 // (2 * NBUF))

    def fetch_idx(t, slot):
        a = pltpu.async_copy(
            src_hbm.at[pl.ds(base + t * NBUF, NBUF)], si_v.at[slot],
            isem.at[slot])
        b = pltpu.async_copy(
            dst_hbm.at[pl.ds(base + t * NBUF, NBUF)], di_v.at[slot],
            isem.at[slot])
        return a, b

    i0 = fetch_idx(0, 0)

    with jax.named_scope("agg_zero"):
        zeros = jnp.zeros((L,), jnp.float32)

        def zbody(i, carry):
            for cc in range(D // L):
                rows_v[0, i, pl.ds(cc * L, L)] = zeros
            return carry

        lax.fori_loop(0, CH, zbody, 0)
        for k in range(RPT // CH):
            pltpu.sync_copy(rows_v.at[0], h_sh.at[pl.ds(s * RPT + k * CH, CH)])
        plsc.subcore_barrier()
        for d in i0:
            d.wait()

    def step(t2, carry):
        for slot in range(2):
            t = t2 * 2 + slot
            nxt = 1 - slot

            @pl.when(t + 1 < 2 * nss2)
            def _():
                fetch_idx(t + 1, nxt)

            gds = []
            for b in range(NBUF):
                gds.append(pltpu.async_copy(
                    g_hbm.at[si_v.at[slot, b]], rows_v.at[b], gsem.at[b]))
            sds = []
            for b in range(NBUF):
                gds[b].wait()
                sds.append(pltpu.async_copy(
                    rows_v.at[b], h_sh.at[di_v.at[slot, b]], ssem.at[b],
                    add=True))
            for d in sds:
                d.wait()

            @pl.when(t + 1 < 2 * nss2)
            def _():
                pltpu.make_async_copy(
                    src_hbm.at[pl.ds(0, NBUF)], si_v.at[nxt],
                    isem.at[nxt]).wait()
                pltpu.make_async_copy(
                    dst_hbm.at[pl.ds(0, NBUF)], di_v.at[nxt],
                    isem.at[nxt]).wait()
        return carry

    with jax.named_scope("agg_main"):
        lax.fori_loop(0, nss2, step, 0)
        plsc.subcore_barrier()

    with jax.named_scope("agg_wb"):
        pltpu.sync_copy(h_sh.at[pl.ds(s * RPT, RPT)],
                        out_hbm.at[c, pl.ds(s * RPT, RPT)])


def _tc_finish(hpart, W, b2):
    def body(p_ref, w_ref, b_ref, o_ref):
        bias2 = b_ref[...] + jnp.sum(w_ref[...], axis=1)[None, :]
        o_ref[...] = p_ref[0] + p_ref[1] + bias2

    RB = 1000
    return pl.pallas_call(
        body,
        grid=(N // RB,),
        in_specs=[
            pl.BlockSpec((NC, RB, D), lambda i: (0, i, 0)),
            pl.BlockSpec((D, D), lambda i: (0, 0)),
            pl.BlockSpec((1, D), lambda i: (0, 0)),
        ],
        out_specs=pl.BlockSpec((RB, D), lambda i: (i, 0)),
        out_shape=jax.ShapeDtypeStruct((N, D), jnp.float32),
    )(hpart, W, b2)


def kernel(feature, edge_index, W_lin, b_lin):
    src = edge_index[0].astype(jnp.int32)
    dst = edge_index[1].astype(jnp.int32)
    pad = jnp.full((EPAD - E,), N, dtype=jnp.int32)
    srcc = jnp.concatenate([src, pad]).reshape(TCH, CH)
    dstc = jnp.concatenate([dst, pad]).reshape(TCH, CH)
    featp = jnp.pad(feature, ((0, NP - N), (0, 0)))

    degs_part = _sc_degrees(srcc)
    g = _tc_project(featp, degs_part, W_lin)
    hpart = _sc_aggregate(g, srcc, dstc)
    return _tc_finish(hpart, W_lin, jnp.reshape(b_lin, (1, D)))
